# Initial kernel scaffold; baseline (speedup 1.0000x reference)
import jax
import jax.numpy as jnp
from jax.experimental import pallas as pl


def kernel(x, h, eps_x, eps_h, batch, edge_index, t_graph, W_e1, b_e1, W_e2, b_e2, W_x, W_h, b_h):
    raise NotImplementedError("write your pallas kernel here")


# speedup vs baseline: 2.4285x; 2.4285x over previous
import jax
import jax.numpy as jnp
from jax.experimental import pallas as pl
from jax.experimental.pallas import tpu as pltpu

T = 1000
BETA_MIN = 0.0001
BETA_MAX = 0.02
N = 50000
E = 1600000
NUM_GRAPHS = 128
IN_DIM = 5
COORD = 3
HID = 32
LW_X = 0.5
LW_H = 0.5

EBLK = 32000


def _mlp_body(d_ref, s_ref, w1d_ref, w1s_ref, wdist_ref, b1_ref, w2_ref,
              b2_ref, wp_ref, out_ref):
    dd = d_ref[...]
    ss = s_ref[...]
    cdim = (((0,), (0,)), ((), ()))
    dx = dd[8:11]
    sx = ss[8:11]
    diff = dx - sx
    dist2 = jnp.sum(diff * diff, axis=0, keepdims=True)
    z1 = (jax.lax.dot_general(w1d_ref[...], dd[0:8], cdim,
                              preferred_element_type=jnp.float32)
          + jax.lax.dot_general(w1s_ref[...], ss[0:8], cdim,
                                preferred_element_type=jnp.float32)
          + wdist_ref[...] * dist2 + b1_ref[...])
    a1 = z1 * jax.nn.sigmoid(z1)
    z2 = jax.lax.dot_general(w2_ref[...], a1, cdim,
                             preferred_element_type=jnp.float32) + b2_ref[...]
    m = z2 * jax.nn.sigmoid(z2)
    pw = jax.lax.dot_general(wp_ref[...], m, cdim,
                             preferred_element_type=jnp.float32)
    wx = pw[5:6] * sx
    out_ref[...] = jnp.concatenate(
        [pw, wx, jnp.zeros((5, pw.shape[1]), jnp.float32)], axis=0)


def _edge_mlp(d, s, w1d, w1s, wdist, b1, w2, b2, wp):
    grid = E // EBLK
    blk16 = pl.BlockSpec((16, EBLK), lambda i: (0, i))
    const = lambda shp: pl.BlockSpec(shp, lambda i: (0, 0))
    return pl.pallas_call(
        _mlp_body,
        grid=(grid,),
        in_specs=[
            blk16, blk16,
            const((8, HID)), const((8, HID)), const((HID, 1)),
            const((HID, 1)), const((HID, HID)), const((HID, 1)),
            const((HID, 8)),
        ],
        out_specs=blk16,
        out_shape=jax.ShapeDtypeStruct((16, E), jnp.float32),
    )(d, s, w1d, w1s, wdist, b1, w2, b2, wp)


def _center(pos, batch):
    s = jax.ops.segment_sum(pos, batch, num_segments=NUM_GRAPHS)
    c = jax.ops.segment_sum(jnp.ones((pos.shape[0], 1), pos.dtype), batch,
                            num_segments=NUM_GRAPHS)
    mean = s / jnp.maximum(c, 1.0)
    return pos - mean[batch]


def kernel(x, h, eps_x, eps_h, batch, edge_index, t_graph, W_e1, b_e1, W_e2,
           b_e2, W_x, W_h, b_h):
    beta = jnp.linspace(BETA_MIN, BETA_MAX, T)
    alpha_bar = jnp.cumprod(1.0 - beta)
    t_node = t_graph[batch][:, None]
    eps_x_c = _center(eps_x, batch)
    ab = alpha_bar[t_node]
    sab = jnp.sqrt(ab)
    s1ab = jnp.sqrt(1.0 - ab)
    x_t = sab * x + s1ab * eps_x_c
    h_t = sab * h + s1ab * eps_h
    t_f = t_node.astype(jnp.float32) / T

    tab = jnp.concatenate(
        [h_t.T, t_f.T, jnp.zeros((2, N), jnp.float32), x_t.T,
         jnp.zeros((5, N), jnp.float32)], axis=0)

    src = edge_index[0]
    dst = edge_index[1]
    d_feat = jnp.take(tab, dst, axis=1)
    s_feat = jnp.take(tab, src, axis=1)

    w1d = jnp.zeros((8, HID), jnp.float32)
    w1d = w1d.at[0:IN_DIM].set(W_e1[0:IN_DIM])
    w1d = w1d.at[IN_DIM].set(W_e1[2 * IN_DIM + 1])
    w1s = jnp.zeros((8, HID), jnp.float32)
    w1s = w1s.at[0:IN_DIM].set(W_e1[IN_DIM:2 * IN_DIM])
    wdist = W_e1[2 * IN_DIM].reshape(HID, 1)
    wp = jnp.zeros((HID, 8), jnp.float32)
    wp = wp.at[:, 0:IN_DIM].set(W_h[IN_DIM:])
    wp = wp.at[:, IN_DIM:IN_DIM + 1].set(W_x)

    out = _edge_mlp(d_feat, s_feat, w1d, w1s, wdist,
                    b_e1.reshape(HID, 1), W_e2, b_e2.reshape(HID, 1), wp)

    acc = jnp.zeros((16, N), jnp.float32).at[:, dst].add(out)

    agg5 = acc[0:IN_DIM].T
    sw = acc[IN_DIM]
    swx = acc[8:8 + COORD].T
    eps_th = h_t @ W_h[:IN_DIM] + agg5 + b_h
    eps_tx = x_t * sw[:, None] - swx
    eps_tx = _center(eps_tx, batch)

    mse_x = jnp.mean((eps_x_c - eps_tx) ** 2)
    mse_h = jnp.mean((eps_h - eps_th) ** 2)
    loss_xh = LW_X * mse_x + LW_H * mse_h
    return (loss_xh, mse_x, mse_h)


# speedup vs baseline: 4.8387x; 1.9925x over previous
import functools

import jax
import jax.numpy as jnp
from jax import lax
from jax.experimental import pallas as pl
from jax.experimental.pallas import tpu as pltpu
from jax.experimental.pallas import tpu_sc as plsc

T = 1000
BETA_MIN = 0.0001
BETA_MAX = 0.02
N = 50000
E = 1600000
NUM_GRAPHS = 128
IN_DIM = 5
COORD = 3
HID = 32
LW_X = 0.5
LW_H = 0.5

NW = 32
EP = 1605632
EPW = EP // NW
CHUNK = 1024
NCH = EPW // CHUNK
NROW = 16
NACC = EPW
TROW = N * NROW // 128
AROW = NACC * NROW // 128
MBLK = 2048


def _wid():
    return lax.axis_index("s") * 2 + lax.axis_index("c")


def _pack_records(bufs_list, big):
    for j in range(len(bufs_list)):
        def row(m, _, j=j):
            for k in range(8):
                big[16 * j + m, pl.ds(16 * k, NROW)] = bufs_list[j][8 * m + k, 0, :]
            return 0
        lax.fori_loop(0, 16, row, 0)


def _unpack_records(big, bufs_list):
    for j in range(len(bufs_list)):
        def row(m, _, j=j):
            for k in range(8):
                bufs_list[j][8 * m + k, 0, :] = big[16 * j + m, pl.ds(16 * k, NROW)]
            return 0
        lax.fori_loop(0, 16, row, 0)


def _gather_body(tab_ref, dst_ref, src_ref, outd_ref, outs_ref,
                 idxd, idxs, bufd, bigd, sem):
    tab_sp = tab_ref
    base = _wid() * EPW
    nr = CHUNK * NROW // 128

    def chunk(ch, _):
        e0 = pl.multiple_of(base + ch * CHUNK, CHUNK)
        pltpu.sync_copy(dst_ref.at[pl.ds(e0, CHUNK)], idxd)
        pltpu.sync_copy(src_ref.at[pl.ds(e0, CHUNK)], idxs)
        r0 = pl.multiple_of(e0 * NROW // 128, 128)
        for idxv, out_ref in ((idxd, outd_ref), (idxs, outs_ref)):
            for w in range(2):
                descs = []
                for jj in range(4):
                    j = 4 * w + jj
                    sl = pl.ds(j * 128, 128)
                    descs.append(pltpu.async_copy(
                        tab_sp.at[idxv.at[sl]], bufd[jj], sem))
                for d in descs:
                    d.wait()
                _pack_records(bufd, bigd)
                pltpu.sync_copy(
                    bigd, out_ref.at[pl.ds(pl.multiple_of(r0 + 64 * w, 64),
                                           nr // 2)])
        return 0

    lax.fori_loop(0, NCH, chunk, 0)


def _sc_gather(tabp, dstg, srcg):
    nsub = CHUNK // 128
    nr = CHUNK * NROW // 128
    fn = pl.kernel(
        _gather_body,
        out_type=(jax.ShapeDtypeStruct((EP * NROW // 128, 128), jnp.float32),
                  jax.ShapeDtypeStruct((EP * NROW // 128, 128), jnp.float32)),
        mesh=plsc.VectorSubcoreMesh(core_axis_name="c", subcore_axis_name="s"),
        scratch_types=[
            pltpu.VMEM((CHUNK,), jnp.int32),
            pltpu.VMEM((CHUNK,), jnp.int32),
            [pltpu.VMEM((128, 1, NROW), jnp.float32) for _ in range(4)],
            pltpu.VMEM((nr // 2, 128), jnp.float32),
            pltpu.SemaphoreType.DMA,
        ],
    )
    return fn(tabp, dstg, srcg)


def _scatter_body(upd_ref, idx_ref, zero_ref, out_ref, ibuf, sbuf, bigu, sem,
                  acc):
    core = lax.axis_index("c")
    sid = lax.axis_index("s")
    wid = sid * 2 + core

    @pl.when(sid == 0)
    def _():
        pltpu.sync_copy(zero_ref, acc)

    plsc.subcore_barrier()
    nr = CHUNK * NROW // 128

    def chunk(ch, _):
        i0 = pl.multiple_of(wid * (NCH * 8) + ch * 8, 8)
        pltpu.sync_copy(idx_ref.at[pl.ds(i0, 8)], ibuf)
        r0 = pl.multiple_of((wid * EPW + ch * CHUNK) * NROW // 128, 128)
        for w in range(2):
            pltpu.sync_copy(
                upd_ref.at[pl.ds(pl.multiple_of(r0 + 64 * w, 64), nr // 2)],
                bigu)
            _unpack_records(bigu, sbuf)
            descs = []
            for jj in range(4):
                descs.append(pltpu.async_copy(
                    sbuf[jj], acc.at[ibuf.at[4 * w + jj]], sem, add=True))
            for d in descs:
                d.wait()
        return 0

    lax.fori_loop(0, NCH, chunk, 0)
    plsc.subcore_barrier()
    rpt = NACC // 16
    pltpu.sync_copy(acc.at[pl.ds(pl.multiple_of(sid * rpt, 8), rpt)],
                    out_ref.at[pl.ds(pl.multiple_of(core * NACC + sid * rpt, 8), rpt)])


def _sc_scatter(updp, idx2d, zerop):
    fn = pl.kernel(
        _scatter_body,
        out_type=jax.ShapeDtypeStruct((2 * NACC, 1, NROW), jnp.float32),
        mesh=plsc.VectorSubcoreMesh(core_axis_name="c", subcore_axis_name="s"),
        scratch_types=[
            pltpu.VMEM((8, 128), jnp.int32),
            [pltpu.VMEM((128, 1, NROW), jnp.float32) for _ in range(4)],
            pltpu.VMEM((CHUNK * NROW // 256, 128), jnp.float32),
            pltpu.SemaphoreType.DMA,
            pltpu.VMEM_SHARED((NACC, 1, NROW), jnp.float32),
        ],
    )
    return fn(updp, idx2d, zerop)


def _mlp_body(d_ref, s_ref, w1d_ref, w1s_ref, q_ref, b1_ref, w2_ref, b2_ref,
              p1_ref, p2_ref, out_ref):
    dd = d_ref[...]
    ss = s_ref[...]
    df = dd - ss
    dsq = df * df
    z1 = (jnp.dot(dd, w1d_ref[...], preferred_element_type=jnp.float32)
          + jnp.dot(ss, w1s_ref[...], preferred_element_type=jnp.float32)
          + jnp.dot(dsq, q_ref[...], preferred_element_type=jnp.float32)
          + b1_ref[...])
    a1 = z1 * jax.nn.sigmoid(z1)
    z2 = jnp.dot(a1, w2_ref[...], preferred_element_type=jnp.float32) + b2_ref[...]
    m = z2 * jax.nn.sigmoid(z2)
    o1 = jnp.dot(m, p1_ref[...], preferred_element_type=jnp.float32)
    ob = jnp.dot(m, p2_ref[...], preferred_element_type=jnp.float32)
    out_ref[...] = o1 + ob * ss


def _edge_mlp(dp, sp, w1d, w1s, q, b1, w2, b2, p1, p2):
    rows = EP * NROW // 128
    grid = rows // MBLK
    blk = pl.BlockSpec((MBLK, 128), lambda i: (i, 0))
    const = lambda shp: pl.BlockSpec(shp, lambda i: (0, 0))
    return pl.pallas_call(
        _mlp_body,
        grid=(grid,),
        in_specs=[
            blk, blk,
            const((128, 256)), const((128, 256)), const((128, 256)),
            const((1, 256)), const((256, 256)), const((1, 256)),
            const((256, 128)), const((256, 128)),
        ],
        out_specs=blk,
        out_shape=jax.ShapeDtypeStruct((rows, 128), jnp.float32),
    )(dp, sp, w1d, w1s, q, b1, w2, b2, p1, p2)


def _block_diag8(w):
    ki, ko = w.shape
    eye = jnp.eye(8, dtype=w.dtype)
    return (eye[:, None, :, None] * w[None, :, None, :]).reshape(8 * ki, 8 * ko)


def _center(pos, batch):
    s = jax.ops.segment_sum(pos, batch, num_segments=NUM_GRAPHS)
    c = jax.ops.segment_sum(jnp.ones((pos.shape[0], 1), pos.dtype), batch,
                            num_segments=NUM_GRAPHS)
    mean = s / jnp.maximum(c, 1.0)
    return pos - mean[batch]


def kernel(x, h, eps_x, eps_h, batch, edge_index, t_graph, W_e1, b_e1, W_e2,
           b_e2, W_x, W_h, b_h):
    beta = jnp.linspace(BETA_MIN, BETA_MAX, T)
    alpha_bar = jnp.cumprod(1.0 - beta)
    t_node = t_graph[batch][:, None]
    eps_x_c = _center(eps_x, batch)
    ab = alpha_bar[t_node]
    sab = jnp.sqrt(ab)
    s1ab = jnp.sqrt(1.0 - ab)
    x_t = sab * x + s1ab * eps_x_c
    h_t = sab * h + s1ab * eps_h
    t_f = t_node.astype(jnp.float32) / T

    tabp = jnp.concatenate(
        [h_t, t_f, jnp.zeros((N, 2), jnp.float32), x_t,
         jnp.zeros((N, 5), jnp.float32)], axis=1).reshape(N, 1, NROW)

    src = edge_index[0].astype(jnp.int32)
    dst = edge_index[1].astype(jnp.int32)
    padz = jnp.zeros((EP - E,), jnp.int32)
    dstg = jnp.concatenate([dst, padz])
    srcg = jnp.concatenate([src, padz])
    paddump = N + (jnp.arange(EP - E, dtype=jnp.int32) % 64)
    idx2d = jnp.concatenate([dst, paddump]).reshape(EP // 128, 128)

    dp, sp = _sc_gather(tabp, dstg, srcg)

    w1db = jnp.zeros((NROW, HID), jnp.float32)
    w1db = w1db.at[0:IN_DIM].set(W_e1[0:IN_DIM])
    w1db = w1db.at[IN_DIM].set(W_e1[2 * IN_DIM + 1])
    w1sb = jnp.zeros((NROW, HID), jnp.float32)
    w1sb = w1sb.at[0:IN_DIM].set(W_e1[IN_DIM:2 * IN_DIM])
    qb = jnp.zeros((NROW, HID), jnp.float32)
    qb = qb.at[8:8 + COORD].set(jnp.tile(W_e1[2 * IN_DIM][None, :], (COORD, 1)))
    p1b = jnp.zeros((HID, NROW), jnp.float32)
    p1b = p1b.at[:, 0:IN_DIM].set(W_h[IN_DIM:])
    p1b = p1b.at[:, IN_DIM:IN_DIM + 1].set(W_x)
    p2b = jnp.zeros((HID, NROW), jnp.float32)
    p2b = p2b.at[:, 8:8 + COORD].set(jnp.tile(W_x, (1, COORD)))

    out = _edge_mlp(
        dp, sp,
        _block_diag8(w1db), _block_diag8(w1sb), _block_diag8(qb),
        jnp.tile(b_e1, 8)[None, :], _block_diag8(W_e2),
        jnp.tile(b_e2, 8)[None, :], _block_diag8(p1b), _block_diag8(p2b))

    upd = out.reshape(EP // 8, 8, NROW).reshape(EP, NROW)
    acc = jnp.zeros((NACC, NROW), jnp.float32).at[
        idx2d.reshape(EP)].add(upd)[:N]

    agg5 = acc[:, 0:IN_DIM]
    sw = acc[:, IN_DIM:IN_DIM + 1]
    swx = acc[:, 8:8 + COORD]
    eps_th = h_t @ W_h[:IN_DIM] + agg5 + b_h
    eps_tx = x_t * sw - swx
    eps_tx = _center(eps_tx, batch)

    mse_x = jnp.mean((eps_x_c - eps_tx) ** 2)
    mse_h = jnp.mean((eps_h - eps_th) ** 2)
    loss_xh = LW_X * mse_x + LW_H * mse_h
    return (loss_xh, mse_x, mse_h)


# speedup vs baseline: 4.8399x; 1.0003x over previous
import functools

import jax
import jax.numpy as jnp
from jax import lax
from jax.experimental import pallas as pl
from jax.experimental.pallas import tpu as pltpu
from jax.experimental.pallas import tpu_sc as plsc

T = 1000
BETA_MIN = 0.0001
BETA_MAX = 0.02
N = 50000
E = 1600000
NUM_GRAPHS = 128
IN_DIM = 5
COORD = 3
HID = 32
LW_X = 0.5
LW_H = 0.5

NW = 32
EP = 1605632
EPW = EP // NW
CHUNK = 1024
NCH = EPW // CHUNK
NROW = 16
NACC = EPW
TROW = N * NROW // 128
AROW = NACC * NROW // 128
MBLK = 2048


def _wid():
    return lax.axis_index("s") * 2 + lax.axis_index("c")


def _pack_records(bufs_list, big):
    for j in range(len(bufs_list)):
        def row(m, _, j=j):
            for k in range(8):
                big[16 * j + m, pl.ds(16 * k, NROW)] = bufs_list[j][8 * m + k, 0, :]
            return 0
        lax.fori_loop(0, 16, row, 0)


def _gather_body(tab_ref, dst_ref, src_ref, outd_ref, outs_ref,
                 idxd, idxs, bufd, bigd, sem):
    tab_sp = tab_ref
    base = _wid() * EPW
    nr = CHUNK * NROW // 128

    def chunk(ch, _):
        e0 = pl.multiple_of(base + ch * CHUNK, CHUNK)
        pltpu.sync_copy(dst_ref.at[pl.ds(e0, CHUNK)], idxd)
        pltpu.sync_copy(src_ref.at[pl.ds(e0, CHUNK)], idxs)
        r0 = pl.multiple_of(e0 * NROW // 128, 128)
        for idxv, out_ref in ((idxd, outd_ref), (idxs, outs_ref)):
            for w in range(2):
                descs = []
                for jj in range(4):
                    j = 4 * w + jj
                    sl = pl.ds(j * 128, 128)
                    descs.append(pltpu.async_copy(
                        tab_sp.at[idxv.at[sl]], bufd[jj], sem))
                for d in descs:
                    d.wait()
                _pack_records(bufd, bigd)
                pltpu.sync_copy(
                    bigd, out_ref.at[pl.ds(pl.multiple_of(r0 + 64 * w, 64),
                                           nr // 2)])
        return 0

    lax.fori_loop(0, NCH, chunk, 0)


def _sc_gather(tabp, dstg, srcg):
    nsub = CHUNK // 128
    nr = CHUNK * NROW // 128
    fn = pl.kernel(
        _gather_body,
        out_type=(jax.ShapeDtypeStruct((EP * NROW // 128, 128), jnp.float32),
                  jax.ShapeDtypeStruct((EP * NROW // 128, 128), jnp.float32)),
        mesh=plsc.VectorSubcoreMesh(core_axis_name="c", subcore_axis_name="s"),
        scratch_types=[
            pltpu.VMEM((CHUNK,), jnp.int32),
            pltpu.VMEM((CHUNK,), jnp.int32),
            [pltpu.VMEM((128, 1, NROW), jnp.float32) for _ in range(4)],
            pltpu.VMEM((nr // 2, 128), jnp.float32),
            pltpu.SemaphoreType.DMA,
        ],
    )
    return fn(tabp, dstg, srcg)


def _mlp_body(d_ref, s_ref, w1d_ref, w1s_ref, q_ref, b1_ref, w2_ref, b2_ref,
              p1_ref, p2_ref, out_ref):
    dd = d_ref[...]
    ss = s_ref[...]
    df = dd - ss
    dsq = df * df
    z1 = (jnp.dot(dd, w1d_ref[...], preferred_element_type=jnp.float32)
          + jnp.dot(ss, w1s_ref[...], preferred_element_type=jnp.float32)
          + jnp.dot(dsq, q_ref[...], preferred_element_type=jnp.float32)
          + b1_ref[...])
    a1 = z1 * jax.nn.sigmoid(z1)
    z2 = jnp.dot(a1, w2_ref[...], preferred_element_type=jnp.float32) + b2_ref[...]
    m = z2 * jax.nn.sigmoid(z2)
    o1 = jnp.dot(m, p1_ref[...], preferred_element_type=jnp.float32)
    ob = jnp.dot(m, p2_ref[...], preferred_element_type=jnp.float32)
    out_ref[...] = o1 + ob * ss


def _edge_mlp(dp, sp, w1d, w1s, q, b1, w2, b2, p1, p2):
    rows = EP * NROW // 128
    grid = rows // MBLK
    blk = pl.BlockSpec((MBLK, 128), lambda i: (i, 0))
    const = lambda shp: pl.BlockSpec(shp, lambda i: (0, 0))
    return pl.pallas_call(
        _mlp_body,
        grid=(grid,),
        in_specs=[
            blk, blk,
            const((128, 256)), const((128, 256)), const((128, 256)),
            const((1, 256)), const((256, 256)), const((1, 256)),
            const((256, 128)), const((256, 128)),
        ],
        out_specs=blk,
        out_shape=jax.ShapeDtypeStruct((rows, 128), jnp.float32),
    )(dp, sp, w1d, w1s, q, b1, w2, b2, p1, p2)


def _block_diag8(w):
    ki, ko = w.shape
    eye = jnp.eye(8, dtype=w.dtype)
    return (eye[:, None, :, None] * w[None, :, None, :]).reshape(8 * ki, 8 * ko)


def _center(pos, batch):
    s = jax.ops.segment_sum(pos, batch, num_segments=NUM_GRAPHS)
    c = jax.ops.segment_sum(jnp.ones((pos.shape[0], 1), pos.dtype), batch,
                            num_segments=NUM_GRAPHS)
    mean = s / jnp.maximum(c, 1.0)
    return pos - mean[batch]


def kernel(x, h, eps_x, eps_h, batch, edge_index, t_graph, W_e1, b_e1, W_e2,
           b_e2, W_x, W_h, b_h):
    beta = jnp.linspace(BETA_MIN, BETA_MAX, T)
    alpha_bar = jnp.cumprod(1.0 - beta)
    t_node = t_graph[batch][:, None]
    eps_x_c = _center(eps_x, batch)
    ab = alpha_bar[t_node]
    sab = jnp.sqrt(ab)
    s1ab = jnp.sqrt(1.0 - ab)
    x_t = sab * x + s1ab * eps_x_c
    h_t = sab * h + s1ab * eps_h
    t_f = t_node.astype(jnp.float32) / T

    tabp = jnp.concatenate(
        [h_t, t_f, jnp.zeros((N, 2), jnp.float32), x_t,
         jnp.zeros((N, 5), jnp.float32)], axis=1).reshape(N, 1, NROW)

    src = edge_index[0].astype(jnp.int32)
    dst = edge_index[1].astype(jnp.int32)
    padz = jnp.zeros((EP - E,), jnp.int32)
    dstg = jnp.concatenate([dst, padz])
    srcg = jnp.concatenate([src, padz])
    paddump = N + (jnp.arange(EP - E, dtype=jnp.int32) % 64)
    idx2d = jnp.concatenate([dst, paddump]).reshape(EP // 128, 128)

    dp, sp = _sc_gather(tabp, dstg, srcg)

    w1db = jnp.zeros((NROW, HID), jnp.float32)
    w1db = w1db.at[0:IN_DIM].set(W_e1[0:IN_DIM])
    w1db = w1db.at[IN_DIM].set(W_e1[2 * IN_DIM + 1])
    w1sb = jnp.zeros((NROW, HID), jnp.float32)
    w1sb = w1sb.at[0:IN_DIM].set(W_e1[IN_DIM:2 * IN_DIM])
    qb = jnp.zeros((NROW, HID), jnp.float32)
    qb = qb.at[8:8 + COORD].set(jnp.tile(W_e1[2 * IN_DIM][None, :], (COORD, 1)))
    p1b = jnp.zeros((HID, NROW), jnp.float32)
    p1b = p1b.at[:, 0:IN_DIM].set(W_h[IN_DIM:])
    p1b = p1b.at[:, IN_DIM:IN_DIM + 1].set(W_x)
    p2b = jnp.zeros((HID, NROW), jnp.float32)
    p2b = p2b.at[:, 8:8 + COORD].set(jnp.tile(W_x, (1, COORD)))

    out = _edge_mlp(
        dp, sp,
        _block_diag8(w1db), _block_diag8(w1sb), _block_diag8(qb),
        jnp.tile(b_e1, 8)[None, :], _block_diag8(W_e2),
        jnp.tile(b_e2, 8)[None, :], _block_diag8(p1b), _block_diag8(p2b))

    upd = out.reshape(EP, NROW)
    acc = jnp.zeros((NACC, NROW), jnp.float32).at[
        idx2d.reshape(EP)].add(upd)[:N]

    agg5 = acc[:, 0:IN_DIM]
    sw = acc[:, IN_DIM:IN_DIM + 1]
    swx = acc[:, 8:8 + COORD]
    eps_th = h_t @ W_h[:IN_DIM] + agg5 + b_h
    eps_tx = x_t * sw - swx
    eps_tx = _center(eps_tx, batch)

    mse_x = jnp.mean((eps_x_c - eps_tx) ** 2)
    mse_h = jnp.mean((eps_h - eps_th) ** 2)
    loss_xh = LW_X * mse_x + LW_H * mse_h
    return (loss_xh, mse_x, mse_h)
